# Initial kernel scaffold; baseline (speedup 1.0000x reference)
import jax
import jax.numpy as jnp
from jax.experimental import pallas as pl


def kernel(mem, idx, shape_raw, val):
    raise NotImplementedError("write your pallas kernel here")


# speedup vs baseline: 5.3050x; 5.3050x over previous
import functools

import jax
import jax.numpy as jnp
from jax import lax
from jax.experimental import pallas as pl
from jax.experimental.pallas import tpu as pltpu
from jax.experimental.pallas import tpu_sc as plsc

_NC = 2
_NS = 16
_NW = _NC * _NS

_FUSE_BLK = 2048


def _fuse_body(shape_ref, val_ref, o_ref):
    sp_col = jax.nn.softplus(shape_ref[...] - 1.0)
    d = val_ref.shape[1]
    pad = jnp.zeros((_FUSE_BLK, 127 - d), jnp.float32)
    o_ref[...] = jnp.concatenate([sp_col, val_ref[...], pad], axis=1)


@functools.lru_cache(maxsize=None)
def _make_scatter(M, N, D):
    ppw = N // _NW
    blk = min(128, ppw)
    chunks = ppw // blk
    mesh = plsc.VectorSubcoreMesh(core_axis_name="c", subcore_axis_name="s",
                                  num_cores=_NC, num_subcores=_NS)

    @functools.partial(
        pl.kernel,
        mesh=mesh,
        scratch_types=[
            pltpu.VMEM((blk,), jnp.int32),
            pltpu.VMEM((blk, 128), jnp.float32),
        ],
    )
    def scatter(idx_hbm, rows_hbm, out_hbm, idx_v, rows_v):
        wid = lax.axis_index("s") * _NC + lax.axis_index("c")
        base = wid * ppw

        def chunk(i, carry):
            p0 = pl.multiple_of(base + i * blk, blk)
            pltpu.sync_copy(idx_hbm.at[pl.ds(p0, blk)], idx_v)
            pltpu.sync_copy(rows_hbm.at[pl.ds(p0, blk)], rows_v)
            pltpu.sync_copy(rows_v, out_hbm.at[idx_v])
            return carry

        lax.fori_loop(0, chunks, chunk, 0)

    return scatter


def kernel(mem, idx, shape_raw, val):
    M, D = mem.shape
    N = idx.shape[0]
    nb = N // _FUSE_BLK
    fused = pl.pallas_call(
        _fuse_body,
        grid=(nb,),
        in_specs=[
            pl.BlockSpec((_FUSE_BLK, 1), lambda b: (b, 0)),
            pl.BlockSpec((_FUSE_BLK, D), lambda b: (b, 0)),
        ],
        out_specs=pl.BlockSpec((_FUSE_BLK, 128), lambda b: (b, 0)),
        out_shape=jax.ShapeDtypeStruct((N, 128), jnp.float32),
    )(shape_raw, val)
    out_ref = jax.new_ref(jnp.zeros((M, 128), jnp.float32))
    _make_scatter(M, N, D)(idx, fused, out_ref)
    return out_ref[...][:, : D + 1]


# speedup vs baseline: 5.8729x; 1.1071x over previous
import functools

import jax
import jax.numpy as jnp
from jax import lax
from jax.experimental import pallas as pl
from jax.experimental.pallas import tpu as pltpu
from jax.experimental.pallas import tpu_sc as plsc

_NC = 2
_NS = 16
_NW = _NC * _NS

_FUSE_BLK = 2048


def _fuse_body(shape_ref, val_ref, o_ref):
    sp_col = jax.nn.softplus(shape_ref[...] - 1.0)
    d = val_ref.shape[1]
    pad = jnp.zeros((_FUSE_BLK, 127 - d), jnp.float32)
    o_ref[...] = jnp.concatenate([sp_col, val_ref[...], pad], axis=1)


@functools.lru_cache(maxsize=None)
def _make_scatter(M, N, D):
    ppw = N // _NW
    blk = min(128, ppw)
    chunks = ppw // blk
    mesh = plsc.VectorSubcoreMesh(core_axis_name="c", subcore_axis_name="s",
                                  num_cores=_NC, num_subcores=_NS)

    nbuf = 4
    assert chunks % nbuf == 0 and chunks // nbuf >= 2

    @functools.partial(
        pl.kernel,
        mesh=mesh,
        scratch_types=[
            [pltpu.VMEM((blk,), jnp.int32) for _ in range(nbuf)],
            [pltpu.VMEM((blk, 128), jnp.float32) for _ in range(nbuf)],
            [pltpu.SemaphoreType.DMA for _ in range(nbuf)],
            [pltpu.SemaphoreType.DMA for _ in range(nbuf)],
        ],
    )
    def scatter(idx_hbm, rows_hbm, out_hbm, idx_vs, rows_vs, in_sems, out_sems):
        wid = lax.axis_index("s") * _NC + lax.axis_index("c")
        base = wid * ppw

        def start_load(c, j):
            p0 = pl.multiple_of(base + c * blk, blk)
            pltpu.async_copy(idx_hbm.at[pl.ds(p0, blk)], idx_vs[j], in_sems[j])
            pltpu.async_copy(rows_hbm.at[pl.ds(p0, blk)], rows_vs[j],
                             in_sems[j])

        def wait_load(c, j):
            p0 = pl.multiple_of(base + c * blk, blk)
            pltpu.make_async_copy(idx_hbm.at[pl.ds(p0, blk)], idx_vs[j],
                                  in_sems[j]).wait()
            pltpu.make_async_copy(rows_hbm.at[pl.ds(p0, blk)], rows_vs[j],
                                  in_sems[j]).wait()

        def start_scatter(j):
            pltpu.async_copy(rows_vs[j], out_hbm.at[idx_vs[j]], out_sems[j])

        def wait_scatter(j):
            pltpu.make_async_copy(rows_vs[j], out_hbm.at[idx_vs[j]],
                                  out_sems[j]).wait()

        for j in range(nbuf):
            start_load(j, j)

        def group(g, carry):
            c0 = g * nbuf
            for j in range(nbuf):
                wait_load(c0 + j, j)
                start_scatter(j)
            nxt = c0 + nbuf

            @pl.when(nxt < chunks)
            def _():
                for j in range(nbuf):
                    wait_scatter(j)
                    start_load(nxt + j, j)

            return carry

        lax.fori_loop(0, chunks // nbuf, group, 0)
        for j in range(nbuf):
            wait_scatter(j)

    return scatter


def kernel(mem, idx, shape_raw, val):
    M, D = mem.shape
    N = idx.shape[0]
    nb = N // _FUSE_BLK
    fused = pl.pallas_call(
        _fuse_body,
        grid=(nb,),
        in_specs=[
            pl.BlockSpec((_FUSE_BLK, 1), lambda b: (b, 0)),
            pl.BlockSpec((_FUSE_BLK, D), lambda b: (b, 0)),
        ],
        out_specs=pl.BlockSpec((_FUSE_BLK, 128), lambda b: (b, 0)),
        out_shape=jax.ShapeDtypeStruct((N, 128), jnp.float32),
    )(shape_raw, val)
    out_ref = jax.new_ref(jnp.zeros((M, 128), jnp.float32))
    _make_scatter(M, N, D)(idx, fused, out_ref)
    return out_ref[...][:, : D + 1]


# speedup vs baseline: 6.5618x; 1.1173x over previous
import functools

import jax
import jax.numpy as jnp
from jax import lax
from jax.experimental import pallas as pl
from jax.experimental.pallas import tpu as pltpu
from jax.experimental.pallas import tpu_sc as plsc

_NC = 2
_NS = 16
_NW = _NC * _NS
_L = 16


def _softplus_body(x_ref, o_ref):
    o_ref[...] = jax.nn.softplus(x_ref[...] - 1.0)


@functools.lru_cache(maxsize=None)
def _make_scatter(M, N, D):
    ppw = N // _NW
    blk = min(128, ppw)
    chunks = ppw // blk
    nbuf = 4
    assert chunks % nbuf == 0 and chunks // nbuf >= 2
    mesh = plsc.VectorSubcoreMesh(core_axis_name="c", subcore_axis_name="s",
                                  num_cores=_NC, num_subcores=_NS)

    @functools.partial(
        pl.kernel,
        mesh=mesh,
        compiler_params=pltpu.CompilerParams(needs_layout_passes=False),
        scratch_types=[
            [pltpu.VMEM((blk,), jnp.int32) for _ in range(nbuf)],
            [pltpu.VMEM((1, blk), jnp.float32) for _ in range(nbuf)],
            [pltpu.VMEM((D, blk), jnp.float32) for _ in range(nbuf)],
            [pltpu.VMEM((blk, 128), jnp.float32) for _ in range(nbuf)],
            [pltpu.SemaphoreType.DMA for _ in range(nbuf)],
            [pltpu.SemaphoreType.DMA for _ in range(nbuf)],
        ],
    )
    def scatter(idx_hbm, sp_hbm, vt_hbm, out_hbm,
                idx_vs, sp_vs, slab_vs, rows_vs, in_sems, out_sems):
        wid = lax.axis_index("s") * _NC + lax.axis_index("c")
        base = wid * ppw

        def load_descs(c, j):
            p0 = pl.multiple_of(base + c * blk, blk)
            r0 = p0 // blk
            return (
                pltpu.make_async_copy(idx_hbm.at[pl.ds(p0, blk)], idx_vs[j],
                                      in_sems[j]),
                pltpu.make_async_copy(sp_hbm.at[pl.ds(r0, 1), :], sp_vs[j],
                                      in_sems[j]),
                pltpu.make_async_copy(vt_hbm.at[:, pl.ds(p0, blk)], slab_vs[j],
                                      in_sems[j]),
            )

        def start_load(c, j):
            for d in load_descs(c, j):
                d.start()

        def wait_load(c, j):
            for d in load_descs(c, j):
                d.wait()

        def scatter_desc(j):
            return pltpu.make_async_copy(rows_vs[j], out_hbm.at[idx_vs[j]],
                                         out_sems[j])

        def assemble(j):
            rows_v = rows_vs[j]
            slab_v = slab_vs[j]
            sp_v = sp_vs[j]
            lanes = lax.iota(jnp.int32, _L)
            for g in range(blk // _L):
                plsc.store_scatter(
                    rows_v,
                    [lanes + g * _L, jnp.zeros((_L,), jnp.int32)],
                    sp_v[0, pl.ds(g * _L, _L)])

            def col(c, carry):
                cvec = jnp.full((_L,), c + 1, jnp.int32)
                for g in range(blk // _L):
                    plsc.store_scatter(
                        rows_v,
                        [lanes + g * _L, cvec],
                        slab_v[c, pl.ds(g * _L, _L)])
                return carry

            lax.fori_loop(0, D, col, 0)

        for j in range(nbuf):
            start_load(j, j)

        def group(g, carry):
            c0 = g * nbuf
            for j in range(nbuf):
                wait_load(c0 + j, j)
                assemble(j)
                scatter_desc(j).start()
            nxt = c0 + nbuf

            @pl.when(nxt < chunks)
            def _():
                for j in range(nbuf):
                    scatter_desc(j).wait()
                    start_load(nxt + j, j)

            return carry

        lax.fori_loop(0, chunks // nbuf, group, 0)
        for j in range(nbuf):
            scatter_desc(j).wait()

    return scatter


def kernel(mem, idx, shape_raw, val):
    M, D = mem.shape
    N = idx.shape[0]
    sp2d = pl.pallas_call(
        _softplus_body,
        out_shape=jax.ShapeDtypeStruct((N // 128, 128), jnp.float32),
    )(shape_raw.reshape(N // 128, 128))
    vt = val.T
    out_ref = jax.new_ref(jnp.zeros((M, 128), jnp.float32))
    _make_scatter(M, N, D)(idx, sp2d, vt, out_ref)
    return out_ref[...][:, : D + 1]
